# Initial kernel scaffold; baseline (speedup 1.0000x reference)
import jax
import jax.numpy as jnp
from jax.experimental import pallas as pl


def kernel(video_feat, tag_feat, tag_embed, bn_v_w, bn_v_b, Wv, bv, bn_t_w, bn_t_b, Wt, bt, W_gat_t2v, attn_l_t2v, attn_r_t2v, Wfc_t2v, bfc_t2v, W_gat_v2v, attn_l_v2v, attn_r_v2v, Wfc_v2v, bfc_v2v, sem_W1, sem_b1, sem_W2, tag_nids, t2v_src, t2v_dst, v2v_src, v2v_dst):
    raise NotImplementedError("write your pallas kernel here")


# speedup vs baseline: 25.7674x; 25.7674x over previous
import functools

import jax
import jax.numpy as jnp
import numpy as np
from jax import lax
from jax.experimental import pallas as pl
from jax.experimental.pallas import tpu as pltpu
from jax.experimental.pallas import tpu_sc as plsc

N_DST = 10000
ND_PAD = 10240
NDEN = 1280
HID = 128
H = 4
DH = 32
NEG_SLOPE = 0.2
BN_SCALE = float(1.0 / np.sqrt(1.0 + 1e-5))
HSW = 256
CH = 48
CHG = 128
NW = 32


def _mesh():
  return plsc.VectorSubcoreMesh(core_axis_name="c", subcore_axis_name="s")


def _pad_to(x, n, fill):
  if x.shape[0] == n:
    return x
  return jnp.concatenate(
      [x, jnp.full((n - x.shape[0],) + x.shape[1:], fill, x.dtype)], 0)


def _sc_gather_rows(table, idx, n_rows):
  n_pad = ((n_rows + NW * CHG - 1) // (NW * CHG)) * (NW * CHG)
  n_chunks = n_pad // CHG
  per_w = n_chunks // NW
  idx3d = _pad_to(idx, n_pad, 0).reshape(n_chunks, 1, CHG)
  d = table.shape[1]

  @functools.partial(
      pl.kernel,
      mesh=_mesh(),
      out_type=jax.ShapeDtypeStruct((n_chunks, CHG, d), jnp.float32),
      scratch_types=[
          pltpu.VMEM((1, CHG), jnp.int32),
          pltpu.VMEM((CHG, d), jnp.float32),
          pltpu.SemaphoreType.DMA,
      ],
  )
  def k(tab_hbm, idx_hbm, out_hbm, idx_v, rows_v, sem):
    c = lax.axis_index("c")
    s = lax.axis_index("s")
    wid = s * 2 + c

    def body(j, carry):
      r = wid * per_w + j
      pltpu.sync_copy(idx_hbm.at[r], idx_v)
      pltpu.async_copy(tab_hbm.at[idx_v.at[0]], rows_v, sem).wait()
      pltpu.sync_copy(rows_v, out_hbm.at[r])
      return carry

    lax.fori_loop(0, per_w, body, 0)

  return k(table, idx3d).reshape(n_pad, d)[:n_rows]


def _sc_edge_pair(hs_t, er_t, src_t, dst_t, ne_t, hs_v, er_v, src_v, dst_v,
                  ne_v):
  def prep(src, dst, n_edges):
    n_pad = ((n_edges + NW * CH - 1) // (NW * CH)) * (NW * CH)
    n_chunks = n_pad // CH
    return (_pad_to(src, n_pad, 0).reshape(n_chunks, 1, CH),
            _pad_to(dst, n_pad, N_DST).reshape(n_chunks, 1, CH),
            n_chunks // NW)

  src3_t, dst3_t, perw_t = prep(src_t, dst_t, ne_t)
  src3_v, dst3_v, perw_v = prep(src_v, dst_v, ne_v)
  NZ_FULL = ND_PAD // 64
  DZ_FULL = NDEN // 64

  @functools.partial(
      pl.kernel,
      mesh=_mesh(),
      compiler_params=pltpu.CompilerParams(needs_layout_passes=False),
      out_type=[
          jax.ShapeDtypeStruct((2, 2, ND_PAD, HID), jnp.float32),
          jax.ShapeDtypeStruct((2, 2, NDEN, HID), jnp.float32),
      ],
      scratch_types=[
          pltpu.VMEM((1, CH), jnp.int32),
          pltpu.VMEM((1, CH), jnp.int32),
          pltpu.VMEM((1, CH), jnp.int32),
          pltpu.VMEM((CH, HSW), jnp.float32),
          pltpu.VMEM((CH, HID), jnp.float32),
          pltpu.VMEM((64, HID), jnp.float32),
          pltpu.VMEM((CH, HID), jnp.float32),
          pltpu.VMEM((1, 16), jnp.float32),
          pltpu.VMEM_SHARED((ND_PAD, HID), jnp.float32),
          pltpu.VMEM_SHARED((NDEN, HID), jnp.float32),
          pltpu.SemaphoreType.DMA,
      ],
  )
  def k(hst_hbm, ert_hbm, srct_hbm, dstt_hbm, hsv_hbm, erv_hbm, srcv_hbm,
        dstv_hbm, num_hbm, den_hbm,
        sidx, didx, didx8, hsb, errb, msgb, denb, wtmp, accn, accd, sem):
    c = lax.axis_index("c")
    s = lax.axis_index("s")
    wid = s * 2 + c
    zv = jnp.zeros((16,), jnp.float32)
    eight = jnp.full((16,), 8, jnp.int32)
    slope = jnp.full((16,), NEG_SLOPE, jnp.float32)

    def splat_i(v):
      return jnp.full((16,), v, jnp.int32)

    def zero_buf(buf, n):
      def zrow(i, carry):
        for kk in range(HID // 16):
          buf[i, pl.ds(kk * 16, 16)] = zv
        return carry

      lax.fori_loop(0, n, zrow, 0)

    def zero_accs():
      zero_buf(msgb, 64)

      def zchunk(it, carry):
        t = s + 16 * it

        @pl.when(t < NZ_FULL)
        def _():
          pltpu.sync_copy(msgb, accn.at[pl.ds(t * 64, 64)])

        @pl.when(t < DZ_FULL)
        def _():
          pltpu.sync_copy(msgb, accd.at[pl.ds(t * 64, 64)])

        return carry

      lax.fori_loop(0, (NZ_FULL + 15) // 16, zchunk, 0)

    def run_etype(hs_hbm, er_hbm, src_hbm, dst_hbm, per_w):
      def step(r, carry):
        row = wid * per_w + r
        pltpu.sync_copy(src_hbm.at[row], sidx)
        pltpu.sync_copy(dst_hbm.at[row], didx)
        pltpu.async_copy(hs_hbm.at[sidx.at[0]], hsb, sem).wait()
        pltpu.async_copy(er_hbm.at[didx.at[0]], errb, sem).wait()

        for g in range(CH // 16):
          dv = didx[0, pl.ds(g * 16, 16)]
          didx8[0, pl.ds(g * 16, 16)] = lax.div(dv, eight)
          for j in range(16):
            i = g * 16 + j
            dj = dv[j]
            ev = hsb[i, pl.ds(HID, 16)] + errb[i, pl.ds(0, 16)]
            w = jnp.exp(jnp.maximum(ev, slope * ev))
            wtmp[0, :] = w
            for kk in range(HID // 16):
              wv = plsc.load_gather(wtmp, [splat_i(0), splat_i(kk // 2)])
              msgb[i, pl.ds(kk * 16, 16)] = hsb[i, pl.ds(kk * 16, 16)] * wv
            denb[i, pl.ds((dj % 8) * 16, 16)] = w

        pltpu.sync_copy(msgb.at[pl.ds(0, CH)], accn.at[didx.at[0]], add=True)
        pltpu.sync_copy(denb, accd.at[didx8.at[0]], add=True)

        for g in range(CH // 16):
          dv = didx[0, pl.ds(g * 16, 16)]
          for j in range(16):
            denb[g * 16 + j, pl.ds((dv[j] % 8) * 16, 16)] = zv
        return carry

      lax.fori_loop(0, per_w, step, 0)

    def writeout(e):
      def wchunk(it, carry):
        t = s + 16 * it

        @pl.when(t < NZ_FULL)
        def _():
          pltpu.sync_copy(accn.at[pl.ds(t * 64, 64)], msgb)
          pltpu.sync_copy(msgb, num_hbm.at[e, c, pl.ds(t * 64, 64)])

        @pl.when(t < DZ_FULL)
        def _():
          pltpu.sync_copy(accd.at[pl.ds(t * 64, 64)], msgb)
          pltpu.sync_copy(msgb, den_hbm.at[e, c, pl.ds(t * 64, 64)])

        return carry

      lax.fori_loop(0, (NZ_FULL + 15) // 16, wchunk, 0)

    zero_accs()
    zero_buf(denb, CH)
    plsc.subcore_barrier()
    run_etype(hst_hbm, ert_hbm, srct_hbm, dstt_hbm, perw_t)
    plsc.subcore_barrier()
    writeout(0)
    plsc.subcore_barrier()
    zero_accs()
    plsc.subcore_barrier()
    run_etype(hsv_hbm, erv_hbm, srcv_hbm, dstv_hbm, perw_v)
    plsc.subcore_barrier()
    writeout(1)

  return k(hs_t, er_t, src3_t, dst3_t, hs_v, er_v, src3_v, dst3_v)


def _sel16():
  col = lax.broadcasted_iota(jnp.int32, (HID, 16), 0)
  hh = lax.broadcasted_iota(jnp.int32, (HID, 16), 1)
  return jnp.where((col // DH) == hh, 1.0, 0.0).astype(jnp.float32)


def _src_dense(x, add_feat, bn_w, bn_b, W, b, Wg, al, blk):
  n = x.shape[0]
  grid = n // blk
  have_add = add_feat is not None

  def body(*refs):
    if have_add:
      (x_ref, a_ref, bw_ref, bb_ref, w_ref, b_ref, wg_ref, al_ref,
       h_ref, hx_ref) = refs
      xx = x_ref[...] + a_ref[...]
    else:
      (x_ref, bw_ref, bb_ref, w_ref, b_ref, wg_ref, al_ref,
       h_ref, hx_ref) = refs
      xx = x_ref[...]
    xb = xx * (bw_ref[...] * BN_SCALE) + bb_ref[...]
    h = jnp.dot(xb, w_ref[...], preferred_element_type=jnp.float32) + b_ref[...]
    hs = jnp.dot(h, wg_ref[...], preferred_element_type=jnp.float32)
    h_ref[...] = h
    el = jnp.dot(hs * al_ref[...], _sel16(), preferred_element_type=jnp.float32)
    hx_ref[...] = jnp.concatenate(
        [hs, el, jnp.zeros((hs.shape[0], HSW - HID - 16), jnp.float32)], 1)

  row_spec = pl.BlockSpec((blk, HID), lambda i: (i, 0))
  full = lambda shape: pl.BlockSpec(shape, lambda i: (0, 0))
  in_specs = [row_spec]
  ins = [x]
  if have_add:
    in_specs.append(row_spec)
    ins.append(add_feat)
  in_specs += [full((1, HID)), full((1, HID)), full((HID, HID)),
               full((1, HID)), full((HID, HID)), full((1, HID))]
  ins += [bn_w.reshape(1, HID), bn_b.reshape(1, HID), W, b.reshape(1, HID),
          Wg, al.reshape(1, HID)]
  return pl.pallas_call(
      body,
      grid=(grid,),
      in_specs=in_specs,
      out_specs=[row_spec, pl.BlockSpec((blk, HSW), lambda i: (i, 0))],
      out_shape=[
          jax.ShapeDtypeStruct((n, HID), jnp.float32),
          jax.ShapeDtypeStruct((n, HSW), jnp.float32),
      ],
  )(*ins)


def _dst_dense(vdst, Wg_t, ar_t, Wfc_t, bfc_t, Wg_v, ar_v, Wfc_v, bfc_v):
  def body(x_ref, wgt_ref, art_ref, wft_ref, bft_ref,
           wgv_ref, arv_ref, wfv_ref, bfv_ref,
           ert_ref, erv_ref, fct_ref, fcv_ref):
    x = x_ref[...]
    col = lax.broadcasted_iota(jnp.int32, (HID, HID), 0)
    hh = lax.broadcasted_iota(jnp.int32, (HID, HID), 1)
    sel = jnp.where((col // DH) == hh, 1.0, 0.0).astype(jnp.float32)
    hdt = jnp.dot(x, wgt_ref[...], preferred_element_type=jnp.float32)
    ert_ref[...] = jnp.dot(hdt * art_ref[...], sel,
                           preferred_element_type=jnp.float32)
    hdv = jnp.dot(x, wgv_ref[...], preferred_element_type=jnp.float32)
    erv_ref[...] = jnp.dot(hdv * arv_ref[...], sel,
                           preferred_element_type=jnp.float32)
    fct_ref[...] = jnp.dot(x, wft_ref[...],
                           preferred_element_type=jnp.float32) + bft_ref[...]
    fcv_ref[...] = jnp.dot(x, wfv_ref[...],
                           preferred_element_type=jnp.float32) + bfv_ref[...]

  blk = 2000
  grid = N_DST // blk
  row = pl.BlockSpec((blk, HID), lambda i: (i, 0))
  full = lambda shape: pl.BlockSpec(shape, lambda i: (0, 0))
  return pl.pallas_call(
      body,
      grid=(grid,),
      in_specs=[row, full((HID, HID)), full((1, HID)), full((HID, HID)),
                full((1, HID)), full((HID, HID)), full((1, HID)),
                full((HID, HID)), full((1, HID))],
      out_specs=[row, row, row, row],
      out_shape=[
          jax.ShapeDtypeStruct((N_DST, HID), jnp.float32),
          jax.ShapeDtypeStruct((N_DST, HID), jnp.float32),
          jax.ShapeDtypeStruct((N_DST, HID), jnp.float32),
          jax.ShapeDtypeStruct((N_DST, HID), jnp.float32),
      ],
  )(vdst, Wg_t, ar_t.reshape(1, HID), Wfc_t, bfc_t.reshape(1, HID),
    Wg_v, ar_v.reshape(1, HID), Wfc_v, bfc_v.reshape(1, HID))


def _final_blocks(num_t, den_t, num_v, den_v, fc_t, fc_v, sem_W1, sem_b1,
                  sem_W2):
  blk = 1000
  grid = N_DST // blk

  def body(nt_ref, dt_ref, nv_ref, dv_ref, ft_ref, fv_ref,
           w1_ref, b1_ref, w2_ref, ht_ref, hv_ref, ws_ref):
    col = lax.broadcasted_iota(jnp.int32, (16, HID), 1)
    hh = lax.broadcasted_iota(jnp.int32, (16, HID), 0)
    rep = jnp.where((col // DH) == hh, 1.0, 0.0).astype(jnp.float32)

    def etype(n_ref, d_ref, f_ref):
      num = n_ref[0] + n_ref[1]
      den = jnp.dot(d_ref[0] + d_ref[1], rep,
                    preferred_element_type=jnp.float32)
      hm = num / (den + 1e-9)
      return f_ref[...] + jnp.maximum(hm, 0.0)

    h_t = etype(nt_ref, dt_ref, ft_ref)
    h_v = etype(nv_ref, dv_ref, fv_ref)
    ht_ref[...] = h_t
    hv_ref[...] = h_v
    w1 = w1_ref[...]
    b1 = b1_ref[...]
    w2 = w2_ref[...]
    wt = jnp.sum(jnp.dot(jnp.tanh(
        jnp.dot(h_t, w1, preferred_element_type=jnp.float32) + b1), w2,
        preferred_element_type=jnp.float32))
    wv = jnp.sum(jnp.dot(jnp.tanh(
        jnp.dot(h_v, w1, preferred_element_type=jnp.float32) + b1), w2,
        preferred_element_type=jnp.float32))
    sel2 = lax.broadcasted_iota(jnp.int32, (1, 2), 1)
    cur = jnp.where(sel2 == 0, wt, wv)

    @pl.when(pl.program_id(0) == 0)
    def _():
      ws_ref[...] = cur

    @pl.when(pl.program_id(0) != 0)
    def _():
      ws_ref[...] = ws_ref[...] + cur

  rowb = lambda w: pl.BlockSpec((2, blk, w), lambda i: (0, i, 0))
  row = pl.BlockSpec((blk, HID), lambda i: (i, 0))
  full = lambda shape: pl.BlockSpec(shape, lambda i: (0,) * len(shape))
  return pl.pallas_call(
      body,
      grid=(grid,),
      in_specs=[rowb(HID), rowb(16), rowb(HID), rowb(16), row, row,
                full((HID, 128)), full((1, 128)), full((128, 1))],
      out_specs=[row, row, full((1, 2))],
      out_shape=[
          jax.ShapeDtypeStruct((N_DST, HID), jnp.float32),
          jax.ShapeDtypeStruct((N_DST, HID), jnp.float32),
          jax.ShapeDtypeStruct((1, 2), jnp.float32),
      ],
  )(num_t, den_t, num_v, den_v, fc_t, fc_v,
    sem_W1, sem_b1.reshape(1, 128), sem_W2)


def _combine(h_t, h_v, wsum):
  blk = 1000
  grid = N_DST // blk

  def body(ht_ref, hv_ref, ws_ref, out_ref):
    wt = ws_ref[0, 0] / N_DST
    wv = ws_ref[0, 1] / N_DST
    m = jnp.maximum(wt, wv)
    et = jnp.exp(wt - m)
    ev = jnp.exp(wv - m)
    bt = et / (et + ev)
    bv = ev / (et + ev)
    hvid = bt * ht_ref[...] + bv * hv_ref[...]
    nrm = jnp.maximum(
        jnp.sqrt(jnp.sum(hvid * hvid, axis=1, keepdims=True)), 1e-12)
    out_ref[...] = hvid / nrm

  row = pl.BlockSpec((blk, HID), lambda i: (i, 0))
  return pl.pallas_call(
      body,
      grid=(grid,),
      in_specs=[row, row, pl.BlockSpec((1, 2), lambda i: (0, 0))],
      out_specs=row,
      out_shape=jax.ShapeDtypeStruct((N_DST, HID), jnp.float32),
  )(h_t, h_v, wsum)


@jax.jit
def kernel(video_feat, tag_feat, tag_embed, bn_v_w, bn_v_b, Wv, bv,
           bn_t_w, bn_t_b, Wt, bt,
           W_gat_t2v, attn_l_t2v, attn_r_t2v, Wfc_t2v, bfc_t2v,
           W_gat_v2v, attn_l_v2v, attn_r_v2v, Wfc_v2v, bfc_v2v,
           sem_W1, sem_b1, sem_W2,
           tag_nids, t2v_src, t2v_dst, v2v_src, v2v_dst):
  n_tag = tag_feat.shape[0]
  gemb = _sc_gather_rows(tag_embed, tag_nids.astype(jnp.int32), n_tag)

  vh, hx_v = _src_dense(video_feat, None, bn_v_w, bn_v_b, Wv, bv,
                        W_gat_v2v, attn_l_v2v, blk=1000)
  _, hx_t = _src_dense(tag_feat, gemb, bn_t_w, bn_t_b, Wt, bt,
                       W_gat_t2v, attn_l_t2v, blk=1000)

  er_t, er_v, fc_t, fc_v = _dst_dense(
      vh[:N_DST], W_gat_t2v, attn_r_t2v, Wfc_t2v, bfc_t2v,
      W_gat_v2v, attn_r_v2v, Wfc_v2v, bfc_v2v)

  num, den = _sc_edge_pair(
      hx_t, _pad_to(er_t, 10016, 0.0), t2v_src.astype(jnp.int32),
      t2v_dst.astype(jnp.int32), t2v_src.shape[0],
      hx_v, _pad_to(er_v, 10016, 0.0), v2v_src.astype(jnp.int32),
      v2v_dst.astype(jnp.int32), v2v_src.shape[0])

  def unpack(num_e, den_e):
    return (num_e[:, :N_DST, :],
            den_e.reshape(2, NDEN * HID // 16, 16)[:, :N_DST, :])

  nt, dt = unpack(num[0], den[0])
  nv, dv = unpack(num[1], den[1])
  h_t, h_v, wsum = _final_blocks(nt, dt, nv, dv, fc_t, fc_v,
                                 sem_W1, sem_b1, sem_W2)
  return _combine(h_t, h_v, wsum)


# speedup vs baseline: 28.6627x; 1.1124x over previous
import functools

import jax
import jax.numpy as jnp
import numpy as np
from jax import lax
from jax.experimental import pallas as pl
from jax.experimental.pallas import tpu as pltpu
from jax.experimental.pallas import tpu_sc as plsc

N_DST = 10000
ND_PAD = 10240
NDEN = 1280
HID = 128
H = 4
DH = 32
NEG_SLOPE = 0.2
BN_SCALE = float(1.0 / np.sqrt(1.0 + 1e-5))
HSW = 256
CH = 48
CHG = 128
NW = 32


def _mesh():
  return plsc.VectorSubcoreMesh(core_axis_name="c", subcore_axis_name="s")


def _pad_to(x, n, fill):
  if x.shape[0] == n:
    return x
  return jnp.concatenate(
      [x, jnp.full((n - x.shape[0],) + x.shape[1:], fill, x.dtype)], 0)


def _sc_gather_rows(table, idx, n_rows):
  n_pad = ((n_rows + NW * CHG - 1) // (NW * CHG)) * (NW * CHG)
  n_chunks = n_pad // CHG
  per_w = n_chunks // NW
  idx3d = _pad_to(idx, n_pad, 0).reshape(n_chunks, 1, CHG)
  d = table.shape[1]

  @functools.partial(
      pl.kernel,
      mesh=_mesh(),
      out_type=jax.ShapeDtypeStruct((n_chunks, CHG, d), jnp.float32),
      scratch_types=[
          pltpu.VMEM((1, CHG), jnp.int32),
          pltpu.VMEM((CHG, d), jnp.float32),
          pltpu.SemaphoreType.DMA,
      ],
  )
  def k(tab_hbm, idx_hbm, out_hbm, idx_v, rows_v, sem):
    c = lax.axis_index("c")
    s = lax.axis_index("s")
    wid = s * 2 + c

    def body(j, carry):
      r = wid * per_w + j
      pltpu.sync_copy(idx_hbm.at[r], idx_v)
      pltpu.async_copy(tab_hbm.at[idx_v.at[0]], rows_v, sem).wait()
      pltpu.sync_copy(rows_v, out_hbm.at[r])
      return carry

    lax.fori_loop(0, per_w, body, 0)

  return k(table, idx3d).reshape(n_pad, d)[:n_rows]


def _sc_edge_pair(hs_t, er_t, src_t, dst_t, ne_t, hs_v, er_v, src_v, dst_v,
                  ne_v):
  def prep(src, dst, n_edges):
    n_pad = ((n_edges + NW * CH - 1) // (NW * CH)) * (NW * CH)
    n_chunks = n_pad // CH
    return (_pad_to(src, n_pad, 0).reshape(n_chunks, 1, CH),
            _pad_to(dst, n_pad, N_DST).reshape(n_chunks, 1, CH),
            n_chunks // NW)

  src3_t, dst3_t, perw_t = prep(src_t, dst_t, ne_t)
  src3_v, dst3_v, perw_v = prep(src_v, dst_v, ne_v)
  NZ_FULL = ND_PAD // 64
  DZ_FULL = NDEN // 64

  @functools.partial(
      pl.kernel,
      mesh=_mesh(),
      compiler_params=pltpu.CompilerParams(needs_layout_passes=False),
      out_type=[
          jax.ShapeDtypeStruct((2, 2, ND_PAD, HID), jnp.float32),
          jax.ShapeDtypeStruct((2, 2, NDEN, HID), jnp.float32),
      ],
      scratch_types=[
          pltpu.VMEM((1, CH), jnp.int32),
          pltpu.VMEM((1, CH), jnp.int32),
          pltpu.VMEM((1, CH), jnp.int32),
          pltpu.VMEM((CH, HSW), jnp.float32),
          pltpu.VMEM((CH, HID), jnp.float32),
          pltpu.VMEM((64, HID), jnp.float32),
          pltpu.VMEM((CH, HID), jnp.float32),
          pltpu.VMEM((1, 16), jnp.float32),
          pltpu.VMEM_SHARED((ND_PAD, HID), jnp.float32),
          pltpu.VMEM_SHARED((NDEN, HID), jnp.float32),
          pltpu.SemaphoreType.DMA,
      ],
  )
  def k(hst_hbm, ert_hbm, srct_hbm, dstt_hbm, hsv_hbm, erv_hbm, srcv_hbm,
        dstv_hbm, num_hbm, den_hbm,
        sidx, didx, didx8, hsb, errb, msgb, denb, wtmp, accn, accd, sem):
    c = lax.axis_index("c")
    s = lax.axis_index("s")
    wid = s * 2 + c
    zv = jnp.zeros((16,), jnp.float32)
    eight = jnp.full((16,), 8, jnp.int32)
    slope = jnp.full((16,), NEG_SLOPE, jnp.float32)

    def splat_i(v):
      return jnp.full((16,), v, jnp.int32)

    def zero_buf(buf, n):
      def zrow(i, carry):
        for kk in range(HID // 16):
          buf[i, pl.ds(kk * 16, 16)] = zv
        return carry

      lax.fori_loop(0, n, zrow, 0)

    def zero_accs():
      zero_buf(msgb, 64)

      def zchunk(it, carry):
        t = s + 16 * it

        @pl.when(t < NZ_FULL)
        def _():
          pltpu.sync_copy(msgb, accn.at[pl.ds(t * 64, 64)])

        @pl.when(t < DZ_FULL)
        def _():
          pltpu.sync_copy(msgb, accd.at[pl.ds(t * 64, 64)])

        return carry

      lax.fori_loop(0, (NZ_FULL + 15) // 16, zchunk, 0)

    def run_etype(hs_hbm, er_hbm, src_hbm, dst_hbm, per_w):
      def step(r, carry):
        row = wid * per_w + r
        pltpu.sync_copy(src_hbm.at[row], sidx)
        pltpu.sync_copy(dst_hbm.at[row], didx)
        cp_hs = pltpu.async_copy(hs_hbm.at[sidx.at[0]], hsb, sem)
        cp_er = pltpu.async_copy(er_hbm.at[didx.at[0]], errb, sem)
        cp_hs.wait()
        cp_er.wait()

        for g in range(CH // 16):
          dv = didx[0, pl.ds(g * 16, 16)]
          didx8[0, pl.ds(g * 16, 16)] = lax.div(dv, eight)
          for j in range(16):
            i = g * 16 + j
            dj = dv[j]
            ev = hsb[i, pl.ds(HID, 16)] + errb[i, pl.ds(0, 16)]
            w = jnp.exp(jnp.maximum(ev, slope * ev))
            wtmp[0, :] = w
            for kk in range(HID // 16):
              wv = plsc.load_gather(wtmp, [splat_i(0), splat_i(kk // 2)])
              msgb[i, pl.ds(kk * 16, 16)] = hsb[i, pl.ds(kk * 16, 16)] * wv
            denb[i, pl.ds((dj % 8) * 16, 16)] = w

        pltpu.sync_copy(msgb.at[pl.ds(0, CH)], accn.at[didx.at[0]], add=True)
        pltpu.sync_copy(denb, accd.at[didx8.at[0]], add=True)

        for g in range(CH // 16):
          dv = didx[0, pl.ds(g * 16, 16)]
          for j in range(16):
            denb[g * 16 + j, pl.ds((dv[j] % 8) * 16, 16)] = zv
        return carry

      lax.fori_loop(0, per_w, step, 0)

    def writeout(e):
      def wchunk(it, carry):
        t = s + 16 * it

        @pl.when(t < NZ_FULL)
        def _():
          pltpu.sync_copy(accn.at[pl.ds(t * 64, 64)], msgb)
          pltpu.sync_copy(msgb, num_hbm.at[e, c, pl.ds(t * 64, 64)])

        @pl.when(t < DZ_FULL)
        def _():
          pltpu.sync_copy(accd.at[pl.ds(t * 64, 64)], msgb)
          pltpu.sync_copy(msgb, den_hbm.at[e, c, pl.ds(t * 64, 64)])

        return carry

      lax.fori_loop(0, (NZ_FULL + 15) // 16, wchunk, 0)

    zero_accs()
    zero_buf(denb, CH)
    plsc.subcore_barrier()
    run_etype(hst_hbm, ert_hbm, srct_hbm, dstt_hbm, perw_t)
    plsc.subcore_barrier()
    writeout(0)
    plsc.subcore_barrier()
    zero_accs()
    plsc.subcore_barrier()
    run_etype(hsv_hbm, erv_hbm, srcv_hbm, dstv_hbm, perw_v)
    plsc.subcore_barrier()
    writeout(1)

  return k(hs_t, er_t, src3_t, dst3_t, hs_v, er_v, src3_v, dst3_v)


def _sel16():
  col = lax.broadcasted_iota(jnp.int32, (HID, 16), 0)
  hh = lax.broadcasted_iota(jnp.int32, (HID, 16), 1)
  return jnp.where((col // DH) == hh, 1.0, 0.0).astype(jnp.float32)


def _src_dense(x, add_feat, bn_w, bn_b, W, b, Wg, al, blk):
  n = x.shape[0]
  grid = n // blk
  have_add = add_feat is not None

  def body(*refs):
    if have_add:
      (x_ref, a_ref, bw_ref, bb_ref, w_ref, b_ref, wg_ref, al_ref,
       h_ref, hx_ref) = refs
      xx = x_ref[...] + a_ref[...]
    else:
      (x_ref, bw_ref, bb_ref, w_ref, b_ref, wg_ref, al_ref,
       h_ref, hx_ref) = refs
      xx = x_ref[...]
    xb = xx * (bw_ref[...] * BN_SCALE) + bb_ref[...]
    h = jnp.dot(xb, w_ref[...], preferred_element_type=jnp.float32) + b_ref[...]
    hs = jnp.dot(h, wg_ref[...], preferred_element_type=jnp.float32)
    h_ref[...] = h
    el = jnp.dot(hs * al_ref[...], _sel16(), preferred_element_type=jnp.float32)
    hx_ref[...] = jnp.concatenate(
        [hs, el, jnp.zeros((hs.shape[0], HSW - HID - 16), jnp.float32)], 1)

  row_spec = pl.BlockSpec((blk, HID), lambda i: (i, 0))
  full = lambda shape: pl.BlockSpec(shape, lambda i: (0, 0))
  in_specs = [row_spec]
  ins = [x]
  if have_add:
    in_specs.append(row_spec)
    ins.append(add_feat)
  in_specs += [full((1, HID)), full((1, HID)), full((HID, HID)),
               full((1, HID)), full((HID, HID)), full((1, HID))]
  ins += [bn_w.reshape(1, HID), bn_b.reshape(1, HID), W, b.reshape(1, HID),
          Wg, al.reshape(1, HID)]
  return pl.pallas_call(
      body,
      grid=(grid,),
      in_specs=in_specs,
      out_specs=[row_spec, pl.BlockSpec((blk, HSW), lambda i: (i, 0))],
      out_shape=[
          jax.ShapeDtypeStruct((n, HID), jnp.float32),
          jax.ShapeDtypeStruct((n, HSW), jnp.float32),
      ],
  )(*ins)


def _dst_dense(vdst, Wg_t, ar_t, Wfc_t, bfc_t, Wg_v, ar_v, Wfc_v, bfc_v):
  def body(x_ref, wgt_ref, art_ref, wft_ref, bft_ref,
           wgv_ref, arv_ref, wfv_ref, bfv_ref,
           ert_ref, erv_ref, fct_ref, fcv_ref):
    x = x_ref[...]
    col = lax.broadcasted_iota(jnp.int32, (HID, HID), 0)
    hh = lax.broadcasted_iota(jnp.int32, (HID, HID), 1)
    sel = jnp.where((col // DH) == hh, 1.0, 0.0).astype(jnp.float32)
    hdt = jnp.dot(x, wgt_ref[...], preferred_element_type=jnp.float32)
    ert_ref[...] = jnp.dot(hdt * art_ref[...], sel,
                           preferred_element_type=jnp.float32)
    hdv = jnp.dot(x, wgv_ref[...], preferred_element_type=jnp.float32)
    erv_ref[...] = jnp.dot(hdv * arv_ref[...], sel,
                           preferred_element_type=jnp.float32)
    fct_ref[...] = jnp.dot(x, wft_ref[...],
                           preferred_element_type=jnp.float32) + bft_ref[...]
    fcv_ref[...] = jnp.dot(x, wfv_ref[...],
                           preferred_element_type=jnp.float32) + bfv_ref[...]

  blk = 2000
  grid = N_DST // blk
  row = pl.BlockSpec((blk, HID), lambda i: (i, 0))
  full = lambda shape: pl.BlockSpec(shape, lambda i: (0, 0))
  return pl.pallas_call(
      body,
      grid=(grid,),
      in_specs=[row, full((HID, HID)), full((1, HID)), full((HID, HID)),
                full((1, HID)), full((HID, HID)), full((1, HID)),
                full((HID, HID)), full((1, HID))],
      out_specs=[row, row, row, row],
      out_shape=[
          jax.ShapeDtypeStruct((N_DST, HID), jnp.float32),
          jax.ShapeDtypeStruct((N_DST, HID), jnp.float32),
          jax.ShapeDtypeStruct((N_DST, HID), jnp.float32),
          jax.ShapeDtypeStruct((N_DST, HID), jnp.float32),
      ],
  )(vdst, Wg_t, ar_t.reshape(1, HID), Wfc_t, bfc_t.reshape(1, HID),
    Wg_v, ar_v.reshape(1, HID), Wfc_v, bfc_v.reshape(1, HID))


def _final_blocks(num_t, den_t, num_v, den_v, fc_t, fc_v, sem_W1, sem_b1,
                  sem_W2):
  blk = 1000
  grid = N_DST // blk

  def body(nt_ref, dt_ref, nv_ref, dv_ref, ft_ref, fv_ref,
           w1_ref, b1_ref, w2_ref, ht_ref, hv_ref, ws_ref):
    col = lax.broadcasted_iota(jnp.int32, (16, HID), 1)
    hh = lax.broadcasted_iota(jnp.int32, (16, HID), 0)
    rep = jnp.where((col // DH) == hh, 1.0, 0.0).astype(jnp.float32)

    def etype(n_ref, d_ref, f_ref):
      num = n_ref[0] + n_ref[1]
      den = jnp.dot(d_ref[0] + d_ref[1], rep,
                    preferred_element_type=jnp.float32)
      hm = num / (den + 1e-9)
      return f_ref[...] + jnp.maximum(hm, 0.0)

    h_t = etype(nt_ref, dt_ref, ft_ref)
    h_v = etype(nv_ref, dv_ref, fv_ref)
    ht_ref[...] = h_t
    hv_ref[...] = h_v
    w1 = w1_ref[...]
    b1 = b1_ref[...]
    w2 = w2_ref[...]
    wt = jnp.sum(jnp.dot(jnp.tanh(
        jnp.dot(h_t, w1, preferred_element_type=jnp.float32) + b1), w2,
        preferred_element_type=jnp.float32))
    wv = jnp.sum(jnp.dot(jnp.tanh(
        jnp.dot(h_v, w1, preferred_element_type=jnp.float32) + b1), w2,
        preferred_element_type=jnp.float32))
    sel2 = lax.broadcasted_iota(jnp.int32, (1, 2), 1)
    cur = jnp.where(sel2 == 0, wt, wv)

    @pl.when(pl.program_id(0) == 0)
    def _():
      ws_ref[...] = cur

    @pl.when(pl.program_id(0) != 0)
    def _():
      ws_ref[...] = ws_ref[...] + cur

  rowb = lambda w: pl.BlockSpec((2, blk, w), lambda i: (0, i, 0))
  row = pl.BlockSpec((blk, HID), lambda i: (i, 0))
  full = lambda shape: pl.BlockSpec(shape, lambda i: (0,) * len(shape))
  return pl.pallas_call(
      body,
      grid=(grid,),
      in_specs=[rowb(HID), rowb(16), rowb(HID), rowb(16), row, row,
                full((HID, 128)), full((1, 128)), full((128, 1))],
      out_specs=[row, row, full((1, 2))],
      out_shape=[
          jax.ShapeDtypeStruct((N_DST, HID), jnp.float32),
          jax.ShapeDtypeStruct((N_DST, HID), jnp.float32),
          jax.ShapeDtypeStruct((1, 2), jnp.float32),
      ],
  )(num_t, den_t, num_v, den_v, fc_t, fc_v,
    sem_W1, sem_b1.reshape(1, 128), sem_W2)


def _combine(h_t, h_v, wsum):
  blk = 1000
  grid = N_DST // blk

  def body(ht_ref, hv_ref, ws_ref, out_ref):
    wt = ws_ref[0, 0] / N_DST
    wv = ws_ref[0, 1] / N_DST
    m = jnp.maximum(wt, wv)
    et = jnp.exp(wt - m)
    ev = jnp.exp(wv - m)
    bt = et / (et + ev)
    bv = ev / (et + ev)
    hvid = bt * ht_ref[...] + bv * hv_ref[...]
    nrm = jnp.maximum(
        jnp.sqrt(jnp.sum(hvid * hvid, axis=1, keepdims=True)), 1e-12)
    out_ref[...] = hvid / nrm

  row = pl.BlockSpec((blk, HID), lambda i: (i, 0))
  return pl.pallas_call(
      body,
      grid=(grid,),
      in_specs=[row, row, pl.BlockSpec((1, 2), lambda i: (0, 0))],
      out_specs=row,
      out_shape=jax.ShapeDtypeStruct((N_DST, HID), jnp.float32),
  )(h_t, h_v, wsum)


@jax.jit
def kernel(video_feat, tag_feat, tag_embed, bn_v_w, bn_v_b, Wv, bv,
           bn_t_w, bn_t_b, Wt, bt,
           W_gat_t2v, attn_l_t2v, attn_r_t2v, Wfc_t2v, bfc_t2v,
           W_gat_v2v, attn_l_v2v, attn_r_v2v, Wfc_v2v, bfc_v2v,
           sem_W1, sem_b1, sem_W2,
           tag_nids, t2v_src, t2v_dst, v2v_src, v2v_dst):
  n_tag = tag_feat.shape[0]
  gemb = _sc_gather_rows(tag_embed, tag_nids.astype(jnp.int32), n_tag)

  vh, hx_v = _src_dense(video_feat, None, bn_v_w, bn_v_b, Wv, bv,
                        W_gat_v2v, attn_l_v2v, blk=1000)
  _, hx_t = _src_dense(tag_feat, gemb, bn_t_w, bn_t_b, Wt, bt,
                       W_gat_t2v, attn_l_t2v, blk=1000)

  er_t, er_v, fc_t, fc_v = _dst_dense(
      vh[:N_DST], W_gat_t2v, attn_r_t2v, Wfc_t2v, bfc_t2v,
      W_gat_v2v, attn_r_v2v, Wfc_v2v, bfc_v2v)

  num, den = _sc_edge_pair(
      hx_t, _pad_to(er_t, 10016, 0.0), t2v_src.astype(jnp.int32),
      t2v_dst.astype(jnp.int32), t2v_src.shape[0],
      hx_v, _pad_to(er_v, 10016, 0.0), v2v_src.astype(jnp.int32),
      v2v_dst.astype(jnp.int32), v2v_src.shape[0])

  def unpack(num_e, den_e):
    return (num_e[:, :N_DST, :],
            den_e.reshape(2, NDEN * HID // 16, 16)[:, :N_DST, :])

  nt, dt = unpack(num[0], den[0])
  nv, dv = unpack(num[1], den[1])
  h_t, h_v, wsum = _final_blocks(nt, dt, nv, dv, fc_t, fc_v,
                                 sem_W1, sem_b1, sem_W2)
  return _combine(h_t, h_v, wsum)


# speedup vs baseline: 30.9666x; 1.0804x over previous
import functools

import jax
import jax.numpy as jnp
import numpy as np
from jax import lax
from jax.experimental import pallas as pl
from jax.experimental.pallas import tpu as pltpu
from jax.experimental.pallas import tpu_sc as plsc

N_DST = 10000
ND_PAD = 10240
NDEN = 1280
HID = 128
H = 4
DH = 32
NEG_SLOPE = 0.2
BN_SCALE = float(1.0 / np.sqrt(1.0 + 1e-5))
HSW = 256
CH = 48
CHG = 128
NW = 32


def _mesh():
  return plsc.VectorSubcoreMesh(core_axis_name="c", subcore_axis_name="s")


def _pad_to(x, n, fill):
  if x.shape[0] == n:
    return x
  return jnp.concatenate(
      [x, jnp.full((n - x.shape[0],) + x.shape[1:], fill, x.dtype)], 0)


def _sc_gather_rows(table, idx, n_rows):
  n_pad = ((n_rows + NW * CHG - 1) // (NW * CHG)) * (NW * CHG)
  n_chunks = n_pad // CHG
  per_w = n_chunks // NW
  idx3d = _pad_to(idx, n_pad, 0).reshape(n_chunks, 1, CHG)
  d = table.shape[1]

  @functools.partial(
      pl.kernel,
      mesh=_mesh(),
      out_type=jax.ShapeDtypeStruct((n_chunks, CHG, d), jnp.float32),
      scratch_types=[
          pltpu.VMEM((1, CHG), jnp.int32),
          pltpu.VMEM((CHG, d), jnp.float32),
          pltpu.SemaphoreType.DMA,
      ],
  )
  def k(tab_hbm, idx_hbm, out_hbm, idx_v, rows_v, sem):
    c = lax.axis_index("c")
    s = lax.axis_index("s")
    wid = s * 2 + c

    def body(j, carry):
      r = wid * per_w + j
      pltpu.sync_copy(idx_hbm.at[r], idx_v)
      pltpu.async_copy(tab_hbm.at[idx_v.at[0]], rows_v, sem).wait()
      pltpu.sync_copy(rows_v, out_hbm.at[r])
      return carry

    lax.fori_loop(0, per_w, body, 0)

  return k(table, idx3d).reshape(n_pad, d)[:n_rows]


def _sc_edge_pair(hs_t, er_t, src_t, dst_t, ne_t, hs_v, er_v, src_v, dst_v,
                  ne_v):
  def prep(src, dst, n_edges):
    n_pad = ((n_edges + NW * CH - 1) // (NW * CH)) * (NW * CH)
    n_chunks = n_pad // CH
    return (_pad_to(src, n_pad, 0).reshape(n_chunks, 1, CH),
            _pad_to(dst, n_pad, N_DST).reshape(n_chunks, 1, CH),
            n_chunks // NW)

  src3_t, dst3_t, perw_t = prep(src_t, dst_t, ne_t)
  src3_v, dst3_v, perw_v = prep(src_v, dst_v, ne_v)
  NZ_FULL = ND_PAD // 64
  DZ_FULL = NDEN // 64

  @functools.partial(
      pl.kernel,
      mesh=_mesh(),
      compiler_params=pltpu.CompilerParams(needs_layout_passes=False),
      out_type=[
          jax.ShapeDtypeStruct((2, 2, ND_PAD, HID), jnp.float32),
          jax.ShapeDtypeStruct((2, 2, NDEN, HID), jnp.float32),
      ],
      scratch_types=[
          pltpu.VMEM((1, CH), jnp.int32),
          pltpu.VMEM((1, CH), jnp.int32),
          pltpu.VMEM((1, CH), jnp.int32),
          pltpu.VMEM((CH, HSW), jnp.float32),
          pltpu.VMEM((CH, HID), jnp.float32),
          pltpu.VMEM((64, HID), jnp.float32),
          pltpu.VMEM((CH, HID), jnp.float32),
          pltpu.VMEM((1, 16), jnp.float32),
          pltpu.VMEM_SHARED((ND_PAD, HID), jnp.float32),
          pltpu.VMEM_SHARED((NDEN, HID), jnp.float32),
          pltpu.SemaphoreType.DMA,
      ],
  )
  def k(hst_hbm, ert_hbm, srct_hbm, dstt_hbm, hsv_hbm, erv_hbm, srcv_hbm,
        dstv_hbm, num_hbm, den_hbm,
        sidx, didx, didx8, hsb, errb, msgb, denb, wtmp, accn, accd, sem):
    c = lax.axis_index("c")
    s = lax.axis_index("s")
    wid = s * 2 + c
    zv = jnp.zeros((16,), jnp.float32)
    eight = jnp.full((16,), 8, jnp.int32)
    slope = jnp.full((16,), NEG_SLOPE, jnp.float32)

    def splat_i(v):
      return jnp.full((16,), v, jnp.int32)

    def zero_buf(buf, n):
      def zrow(i, carry):
        for kk in range(HID // 16):
          buf[i, pl.ds(kk * 16, 16)] = zv
        return carry

      lax.fori_loop(0, n, zrow, 0)

    def zero_accs():
      zero_buf(msgb, 64)

      def zchunk(it, carry):
        t = s + 16 * it

        @pl.when(t < NZ_FULL)
        def _():
          pltpu.sync_copy(msgb, accn.at[pl.ds(t * 64, 64)])

        @pl.when(t < DZ_FULL)
        def _():
          pltpu.sync_copy(msgb, accd.at[pl.ds(t * 64, 64)])

        return carry

      lax.fori_loop(0, (NZ_FULL + 15) // 16, zchunk, 0)

    def run_etype(hs_hbm, er_hbm, src_hbm, dst_hbm, per_w):
      def step(r, carry):
        row = wid * per_w + r
        pltpu.sync_copy(src_hbm.at[row], sidx)
        pltpu.sync_copy(dst_hbm.at[row], didx)
        cp_hs = pltpu.async_copy(hs_hbm.at[sidx.at[0]], hsb, sem)
        cp_er = pltpu.async_copy(er_hbm.at[didx.at[0]], errb, sem)
        cp_hs.wait()
        cp_er.wait()

        for g in range(CH // 16):
          dv = didx[0, pl.ds(g * 16, 16)]
          didx8[0, pl.ds(g * 16, 16)] = lax.div(dv, eight)
          for j in range(16):
            i = g * 16 + j
            dj = dv[j]
            ev = hsb[i, pl.ds(HID, 16)] + errb[i, pl.ds(0, 16)]
            w = jnp.exp(jnp.maximum(ev, slope * ev))
            wtmp[0, :] = w
            for h4 in range(H):
              wv = plsc.load_gather(wtmp, [splat_i(0), splat_i(h4)])
              msgb[i, pl.ds(h4 * 32, 16)] = hsb[i, pl.ds(h4 * 32, 16)] * wv
              msgb[i, pl.ds(h4 * 32 + 16, 16)] = (
                  hsb[i, pl.ds(h4 * 32 + 16, 16)] * wv)
            denb[i, pl.ds((dj % 8) * 16, 16)] = w

        pltpu.sync_copy(msgb.at[pl.ds(0, CH)], accn.at[didx.at[0]], add=True)
        pltpu.sync_copy(denb, accd.at[didx8.at[0]], add=True)

        for g in range(CH // 16):
          dv = didx[0, pl.ds(g * 16, 16)]
          for j in range(16):
            denb[g * 16 + j, pl.ds((dv[j] % 8) * 16, 16)] = zv
        return carry

      lax.fori_loop(0, per_w, step, 0)

    def writeout(e):
      def wchunk(it, carry):
        t = s + 16 * it

        @pl.when(t < NZ_FULL)
        def _():
          pltpu.sync_copy(accn.at[pl.ds(t * 64, 64)], msgb)
          pltpu.sync_copy(msgb, num_hbm.at[e, c, pl.ds(t * 64, 64)])

        @pl.when(t < DZ_FULL)
        def _():
          pltpu.sync_copy(accd.at[pl.ds(t * 64, 64)], msgb)
          pltpu.sync_copy(msgb, den_hbm.at[e, c, pl.ds(t * 64, 64)])

        return carry

      lax.fori_loop(0, (NZ_FULL + 15) // 16, wchunk, 0)

    zero_accs()
    zero_buf(denb, CH)
    plsc.subcore_barrier()
    run_etype(hst_hbm, ert_hbm, srct_hbm, dstt_hbm, perw_t)
    plsc.subcore_barrier()
    writeout(0)
    plsc.subcore_barrier()
    zero_accs()
    plsc.subcore_barrier()
    run_etype(hsv_hbm, erv_hbm, srcv_hbm, dstv_hbm, perw_v)
    plsc.subcore_barrier()
    writeout(1)

  return k(hs_t, er_t, src3_t, dst3_t, hs_v, er_v, src3_v, dst3_v)


def _sel16():
  col = lax.broadcasted_iota(jnp.int32, (HID, 16), 0)
  hh = lax.broadcasted_iota(jnp.int32, (HID, 16), 1)
  return jnp.where((col // DH) == hh, 1.0, 0.0).astype(jnp.float32)


def _src_dense(x, add_feat, bn_w, bn_b, W, b, Wg, al, blk):
  n = x.shape[0]
  grid = n // blk
  have_add = add_feat is not None

  def body(*refs):
    if have_add:
      (x_ref, a_ref, bw_ref, bb_ref, w_ref, b_ref, wg_ref, al_ref,
       h_ref, hx_ref) = refs
      xx = x_ref[...] + a_ref[...]
    else:
      (x_ref, bw_ref, bb_ref, w_ref, b_ref, wg_ref, al_ref,
       h_ref, hx_ref) = refs
      xx = x_ref[...]
    xb = xx * (bw_ref[...] * BN_SCALE) + bb_ref[...]
    h = jnp.dot(xb, w_ref[...], preferred_element_type=jnp.float32) + b_ref[...]
    hs = jnp.dot(h, wg_ref[...], preferred_element_type=jnp.float32)
    h_ref[...] = h
    el = jnp.dot(hs * al_ref[...], _sel16(), preferred_element_type=jnp.float32)
    hx_ref[...] = jnp.concatenate(
        [hs, el, jnp.zeros((hs.shape[0], HSW - HID - 16), jnp.float32)], 1)

  row_spec = pl.BlockSpec((blk, HID), lambda i: (i, 0))
  full = lambda shape: pl.BlockSpec(shape, lambda i: (0, 0))
  in_specs = [row_spec]
  ins = [x]
  if have_add:
    in_specs.append(row_spec)
    ins.append(add_feat)
  in_specs += [full((1, HID)), full((1, HID)), full((HID, HID)),
               full((1, HID)), full((HID, HID)), full((1, HID))]
  ins += [bn_w.reshape(1, HID), bn_b.reshape(1, HID), W, b.reshape(1, HID),
          Wg, al.reshape(1, HID)]
  return pl.pallas_call(
      body,
      grid=(grid,),
      in_specs=in_specs,
      out_specs=[row_spec, pl.BlockSpec((blk, HSW), lambda i: (i, 0))],
      out_shape=[
          jax.ShapeDtypeStruct((n, HID), jnp.float32),
          jax.ShapeDtypeStruct((n, HSW), jnp.float32),
      ],
  )(*ins)


def _dst_dense(vdst, Wg_t, ar_t, Wfc_t, bfc_t, Wg_v, ar_v, Wfc_v, bfc_v):
  def body(x_ref, wgt_ref, art_ref, wft_ref, bft_ref,
           wgv_ref, arv_ref, wfv_ref, bfv_ref,
           ert_ref, erv_ref, fct_ref, fcv_ref):
    x = x_ref[...]
    col = lax.broadcasted_iota(jnp.int32, (HID, HID), 0)
    hh = lax.broadcasted_iota(jnp.int32, (HID, HID), 1)
    sel = jnp.where((col // DH) == hh, 1.0, 0.0).astype(jnp.float32)
    hdt = jnp.dot(x, wgt_ref[...], preferred_element_type=jnp.float32)
    ert_ref[...] = jnp.dot(hdt * art_ref[...], sel,
                           preferred_element_type=jnp.float32)
    hdv = jnp.dot(x, wgv_ref[...], preferred_element_type=jnp.float32)
    erv_ref[...] = jnp.dot(hdv * arv_ref[...], sel,
                           preferred_element_type=jnp.float32)
    fct_ref[...] = jnp.dot(x, wft_ref[...],
                           preferred_element_type=jnp.float32) + bft_ref[...]
    fcv_ref[...] = jnp.dot(x, wfv_ref[...],
                           preferred_element_type=jnp.float32) + bfv_ref[...]

  blk = 2000
  grid = N_DST // blk
  row = pl.BlockSpec((blk, HID), lambda i: (i, 0))
  full = lambda shape: pl.BlockSpec(shape, lambda i: (0, 0))
  return pl.pallas_call(
      body,
      grid=(grid,),
      in_specs=[row, full((HID, HID)), full((1, HID)), full((HID, HID)),
                full((1, HID)), full((HID, HID)), full((1, HID)),
                full((HID, HID)), full((1, HID))],
      out_specs=[row, row, row, row],
      out_shape=[
          jax.ShapeDtypeStruct((N_DST, HID), jnp.float32),
          jax.ShapeDtypeStruct((N_DST, HID), jnp.float32),
          jax.ShapeDtypeStruct((N_DST, HID), jnp.float32),
          jax.ShapeDtypeStruct((N_DST, HID), jnp.float32),
      ],
  )(vdst, Wg_t, ar_t.reshape(1, HID), Wfc_t, bfc_t.reshape(1, HID),
    Wg_v, ar_v.reshape(1, HID), Wfc_v, bfc_v.reshape(1, HID))


def _final_blocks(num_t, den_t, num_v, den_v, fc_t, fc_v, sem_W1, sem_b1,
                  sem_W2):
  blk = 1000
  grid = N_DST // blk

  def body(nt_ref, dt_ref, nv_ref, dv_ref, ft_ref, fv_ref,
           w1_ref, b1_ref, w2_ref, ht_ref, hv_ref, ws_ref):
    col = lax.broadcasted_iota(jnp.int32, (16, HID), 1)
    hh = lax.broadcasted_iota(jnp.int32, (16, HID), 0)
    rep = jnp.where((col // DH) == hh, 1.0, 0.0).astype(jnp.float32)

    def etype(n_ref, d_ref, f_ref):
      num = n_ref[0] + n_ref[1]
      den = jnp.dot(d_ref[0] + d_ref[1], rep,
                    preferred_element_type=jnp.float32)
      hm = num / (den + 1e-9)
      return f_ref[...] + jnp.maximum(hm, 0.0)

    h_t = etype(nt_ref, dt_ref, ft_ref)
    h_v = etype(nv_ref, dv_ref, fv_ref)
    ht_ref[...] = h_t
    hv_ref[...] = h_v
    w1 = w1_ref[...]
    b1 = b1_ref[...]
    w2 = w2_ref[...]
    wt = jnp.sum(jnp.dot(jnp.tanh(
        jnp.dot(h_t, w1, preferred_element_type=jnp.float32) + b1), w2,
        preferred_element_type=jnp.float32))
    wv = jnp.sum(jnp.dot(jnp.tanh(
        jnp.dot(h_v, w1, preferred_element_type=jnp.float32) + b1), w2,
        preferred_element_type=jnp.float32))
    sel2 = lax.broadcasted_iota(jnp.int32, (1, 2), 1)
    cur = jnp.where(sel2 == 0, wt, wv)

    @pl.when(pl.program_id(0) == 0)
    def _():
      ws_ref[...] = cur

    @pl.when(pl.program_id(0) != 0)
    def _():
      ws_ref[...] = ws_ref[...] + cur

  rowb = lambda w: pl.BlockSpec((2, blk, w), lambda i: (0, i, 0))
  row = pl.BlockSpec((blk, HID), lambda i: (i, 0))
  full = lambda shape: pl.BlockSpec(shape, lambda i: (0,) * len(shape))
  return pl.pallas_call(
      body,
      grid=(grid,),
      in_specs=[rowb(HID), rowb(16), rowb(HID), rowb(16), row, row,
                full((HID, 128)), full((1, 128)), full((128, 1))],
      out_specs=[row, row, full((1, 2))],
      out_shape=[
          jax.ShapeDtypeStruct((N_DST, HID), jnp.float32),
          jax.ShapeDtypeStruct((N_DST, HID), jnp.float32),
          jax.ShapeDtypeStruct((1, 2), jnp.float32),
      ],
  )(num_t, den_t, num_v, den_v, fc_t, fc_v,
    sem_W1, sem_b1.reshape(1, 128), sem_W2)


def _combine(h_t, h_v, wsum):
  blk = 1000
  grid = N_DST // blk

  def body(ht_ref, hv_ref, ws_ref, out_ref):
    wt = ws_ref[0, 0] / N_DST
    wv = ws_ref[0, 1] / N_DST
    m = jnp.maximum(wt, wv)
    et = jnp.exp(wt - m)
    ev = jnp.exp(wv - m)
    bt = et / (et + ev)
    bv = ev / (et + ev)
    hvid = bt * ht_ref[...] + bv * hv_ref[...]
    nrm = jnp.maximum(
        jnp.sqrt(jnp.sum(hvid * hvid, axis=1, keepdims=True)), 1e-12)
    out_ref[...] = hvid / nrm

  row = pl.BlockSpec((blk, HID), lambda i: (i, 0))
  return pl.pallas_call(
      body,
      grid=(grid,),
      in_specs=[row, row, pl.BlockSpec((1, 2), lambda i: (0, 0))],
      out_specs=row,
      out_shape=jax.ShapeDtypeStruct((N_DST, HID), jnp.float32),
  )(h_t, h_v, wsum)


@jax.jit
def kernel(video_feat, tag_feat, tag_embed, bn_v_w, bn_v_b, Wv, bv,
           bn_t_w, bn_t_b, Wt, bt,
           W_gat_t2v, attn_l_t2v, attn_r_t2v, Wfc_t2v, bfc_t2v,
           W_gat_v2v, attn_l_v2v, attn_r_v2v, Wfc_v2v, bfc_v2v,
           sem_W1, sem_b1, sem_W2,
           tag_nids, t2v_src, t2v_dst, v2v_src, v2v_dst):
  n_tag = tag_feat.shape[0]
  gemb = _sc_gather_rows(tag_embed, tag_nids.astype(jnp.int32), n_tag)

  vh, hx_v = _src_dense(video_feat, None, bn_v_w, bn_v_b, Wv, bv,
                        W_gat_v2v, attn_l_v2v, blk=1000)
  _, hx_t = _src_dense(tag_feat, gemb, bn_t_w, bn_t_b, Wt, bt,
                       W_gat_t2v, attn_l_t2v, blk=1000)

  er_t, er_v, fc_t, fc_v = _dst_dense(
      vh[:N_DST], W_gat_t2v, attn_r_t2v, Wfc_t2v, bfc_t2v,
      W_gat_v2v, attn_r_v2v, Wfc_v2v, bfc_v2v)

  num, den = _sc_edge_pair(
      hx_t, _pad_to(er_t, 10016, 0.0), t2v_src.astype(jnp.int32),
      t2v_dst.astype(jnp.int32), t2v_src.shape[0],
      hx_v, _pad_to(er_v, 10016, 0.0), v2v_src.astype(jnp.int32),
      v2v_dst.astype(jnp.int32), v2v_src.shape[0])

  def unpack(num_e, den_e):
    return (num_e[:, :N_DST, :],
            den_e.reshape(2, NDEN * HID // 16, 16)[:, :N_DST, :])

  nt, dt = unpack(num[0], den[0])
  nv, dv = unpack(num[1], den[1])
  h_t, h_v, wsum = _final_blocks(nt, dt, nv, dv, fc_t, fc_v,
                                 sem_W1, sem_b1, sem_W2)
  return _combine(h_t, h_v, wsum)
